# Initial kernel scaffold; baseline (speedup 1.0000x reference)
import jax
import jax.numpy as jnp
from jax.experimental import pallas as pl


def kernel(inputs):
    raise NotImplementedError("write your pallas kernel here")


# speedup vs baseline: 29.2393x; 29.2393x over previous
import jax
import jax.numpy as jnp
from jax.experimental import pallas as pl
from jax.experimental.pallas import tpu as pltpu

_K = 8
_SB = 512
_CB = 128


def _topk_body(x_ref, o_ref, t_ref):
    s = pl.program_id(2)
    ns = pl.num_programs(2)

    neg = jnp.float32(-jnp.inf)
    init = jnp.full((_K, 8, _CB), neg, jnp.float32)
    T = jnp.where(s == 0, init, t_ref[...])
    Ts = [T[j] for j in range(_K)]

    x = x_ref[0]
    for i in range(_SB // 8):
        v = x[i * 8:(i + 1) * 8, :]
        for j in range(_K):
            hi = jnp.maximum(Ts[j], v)
            v = jnp.minimum(Ts[j], v)
            Ts[j] = hi
    t_ref[...] = jnp.stack(Ts)

    @pl.when(s == ns - 1)
    def _():
        cand = jnp.stack(Ts).reshape(_K * 8, _CB)
        rows = jax.lax.broadcasted_iota(jnp.int32, (_K * 8, _CB), 0)
        outs = []
        c = cand
        for j in range(_K):
            m = jnp.max(c, axis=0, keepdims=True)
            outs.append(m)
            if j < _K - 1:
                eq = c == m
                idx = jnp.where(eq, rows, _K * 8)
                amin = jnp.min(idx, axis=0, keepdims=True)
                c = jnp.where(rows == amin, neg, c)
        o_ref[0] = jnp.concatenate(outs, axis=0)


def kernel(inputs):
    B, S, C = inputs.shape
    grid = (B, C // _CB, S // _SB)
    out3 = pl.pallas_call(
        _topk_body,
        grid=grid,
        in_specs=[pl.BlockSpec((1, _SB, _CB), lambda b, c, s: (b, s, c))],
        out_specs=pl.BlockSpec((1, _K, _CB), lambda b, c, s: (b, 0, c)),
        out_shape=jax.ShapeDtypeStruct((B, _K, C), jnp.float32),
        scratch_shapes=[pltpu.VMEM((_K, 8, _CB), jnp.float32)],
        compiler_params=pltpu.CompilerParams(
            dimension_semantics=("parallel", "parallel", "arbitrary")),
    )(inputs)
    return jnp.transpose(out3, (0, 2, 1)).reshape(B, C * _K)
